# baseline (speedup 1.0000x reference)
import functools

import jax
import jax.numpy as jnp
from jax import lax
from jax.experimental import pallas as pl
from jax.experimental.pallas import tpu as pltpu
from jax.experimental.pallas import tpu_sc as plsc

_VOCAB = 1000000
_EMB = 32
_B = 16384
_L = 200
_PAD = 0

_COLS_A = 65536
_GRID_A = -(-_VOCAB // _COLS_A)

_NC, _NS = 2, 16
_NW = _NC * _NS
_COLS_W = _B // _NW
_CH = 128
_NCH = _COLS_W // _CH
_CHW = _L * _CH


def _score_body(x_ref, w_ref, out_ref):
    x = x_ref[...]
    w = w_ref[...]
    s = jnp.sum(x * w, axis=0, keepdims=True)
    i = pl.program_id(0)
    col = lax.broadcasted_iota(jnp.int32, (1, _COLS_A), 1)
    s = jnp.where((i == 0) & (col == _PAD), 0.0, s)
    out_ref[...] = s.reshape(_COLS_A)


def _scores(table_t, w_col):
    return pl.pallas_call(
        _score_body,
        grid=(_GRID_A,),
        in_specs=[
            pl.BlockSpec((_EMB, _COLS_A), lambda i: (0, i)),
            pl.BlockSpec((_EMB, 1), lambda i: (0, 0)),
        ],
        out_specs=pl.BlockSpec((_COLS_A,), lambda i: (i,)),
        out_shape=jax.ShapeDtypeStruct((_VOCAB,), jnp.float32),
    )(table_t, w_col)


def _sc_permute_body(ids_hbm, out_hbm, buf0, buf1, sema, semw):
    wid = lax.axis_index("s") * _NC + lax.axis_index("c")
    base = wid * _COLS_W
    buf = [buf0, buf1]

    def assemble(ci, b):
        col0 = base + ci * _CH

        def cp_issue(l, c2):
            pltpu.async_copy(ids_hbm.at[l, pl.ds(col0, _CH)],
                             b.at[pl.ds(l * _CH, _CH)], sema)
            return c2

        def cp_drain(l, c2):
            pltpu.make_async_copy(ids_hbm.at[l, pl.ds(col0, _CH)],
                                  b.at[pl.ds(l * _CH, _CH)], sema).wait()
            return c2

        lax.fori_loop(0, _L, cp_issue, 0)
        lax.fori_loop(0, _L, cp_drain, 0)

    assemble(0, buf[0])
    for ci in range(_NCH):
        cid = wid * _NCH + ci
        pltpu.async_copy(buf[ci % 2], out_hbm.at[pl.ds(cid * _CHW, _CHW)],
                         semw)
        if ci + 1 < _NCH:
            assemble(ci + 1, buf[(ci + 1) % 2])
        pltpu.make_async_copy(buf[ci % 2],
                              out_hbm.at[pl.ds(cid * _CHW, _CHW)],
                              semw).wait()


def _sc_permute(ids_t):
    mesh = plsc.VectorSubcoreMesh(core_axis_name="c", subcore_axis_name="s")
    f = pl.kernel(
        _sc_permute_body,
        out_type=jax.ShapeDtypeStruct((_B * _L,), jnp.int32),
        mesh=mesh,
        scratch_types=[
            pltpu.VMEM((_CHW,), jnp.int32),
            pltpu.VMEM((_CHW,), jnp.int32),
            pltpu.SemaphoreType.DMA,
            pltpu.SemaphoreType.DMA,
        ],
    )
    return f(ids_t)


def _sc_pool_body(scores_hbm, idsl_hbm, bvec_hbm, out_hbm,
                  idx0, idx1, vals0, vals1, out_v, b_v, semg, sema):
    wid = lax.axis_index("s") * _NC + lax.axis_index("c")
    base = wid * _COLS_W
    pltpu.sync_copy(bvec_hbm, b_v)
    bv = b_v[...]
    idx = [idx0, idx1]
    vals = [vals0, vals1]

    def load_idx(ci, buf):
        cid = wid * _NCH + ci
        pltpu.sync_copy(idsl_hbm.at[pl.ds(cid * _CHW, _CHW)], buf)

    load_idx(0, idx[0])
    pltpu.async_copy(scores_hbm.at[idx[0]], vals[0], semg)
    load_idx(1, idx[1])
    for ci in range(_NCH):
        cur = ci % 2
        pltpu.make_async_copy(scores_hbm.at[idx[cur]], vals[cur], semg).wait()
        if ci + 1 < _NCH:
            pltpu.async_copy(scores_hbm.at[idx[1 - cur]], vals[1 - cur], semg)
        if ci + 2 < _NCH:
            load_idx(ci + 2, idx[cur])

        def red(l, accs):
            base_l = l * _CH
            return tuple(a + vals[cur][pl.ds(base_l + k * 16, 16)]
                         for k, a in enumerate(accs))

        zero = jnp.zeros((16,), jnp.float32)
        accs = lax.fori_loop(0, _L, red, (zero,) * (_CH // 16))
        for k, acc in enumerate(accs):
            z = acc + bv
            y = 1.0 / (1.0 + jnp.exp(-z))
            out_v[pl.ds(ci * _CH + k * 16, 16)] = y

    pltpu.sync_copy(out_v, out_hbm.at[pl.ds(base, _COLS_W)])


def _sc_pool(scores, ids_t, bvec):
    mesh = plsc.VectorSubcoreMesh(core_axis_name="c", subcore_axis_name="s")
    f = pl.kernel(
        _sc_pool_body,
        out_type=jax.ShapeDtypeStruct((_B,), jnp.float32),
        mesh=mesh,
        scratch_types=[
            pltpu.VMEM((_CHW,), jnp.int32),
            pltpu.VMEM((_CHW,), jnp.int32),
            pltpu.VMEM((_CHW,), jnp.float32),
            pltpu.VMEM((_CHW,), jnp.float32),
            pltpu.VMEM((_COLS_W,), jnp.float32),
            pltpu.VMEM((16,), jnp.float32),
            pltpu.SemaphoreType.DMA,
            pltpu.SemaphoreType.DMA,
        ],
    )
    return f(scores, ids_t, bvec)


def kernel(ids, table, W, b):
    table_t = table.astype(jnp.float32).T
    ids_t = ids.astype(jnp.int32).T
    w_col = W.astype(jnp.float32).reshape(_EMB, 1) * (1.0 / _L)
    ids_l = _sc_permute(ids_t)
    scores = _scores(table_t, w_col)
    bvec = jnp.broadcast_to(b.astype(jnp.float32), (16,))
    out_flat = _sc_pool(scores, ids_l, bvec)
    return out_flat.reshape(_B, 1)


# speedup vs baseline: 1.0377x; 1.0377x over previous
import functools

import jax
import jax.numpy as jnp
from jax import lax
from jax.experimental import pallas as pl
from jax.experimental.pallas import tpu as pltpu
from jax.experimental.pallas import tpu_sc as plsc

_VOCAB = 1000000
_EMB = 32
_B = 16384
_L = 200
_PAD = 0

_COLS_A = 65536
_GRID_A = -(-_VOCAB // _COLS_A)

_NC, _NS = 2, 16
_NW = _NC * _NS
_COLS_W = _B // _NW
_CH = 128
_NCH = _COLS_W // _CH
_CHW = _L * _CH


def _score_body(x_ref, w_ref, out_ref):
    x = x_ref[...]
    w = w_ref[...]
    s = jnp.sum(x * w, axis=0, keepdims=True)
    i = pl.program_id(0)
    col = lax.broadcasted_iota(jnp.int32, (1, _COLS_A), 1)
    s = jnp.where((i == 0) & (col == _PAD), 0.0, s)
    out_ref[...] = s.reshape(_COLS_A)


def _scores(table_t, w_col):
    return pl.pallas_call(
        _score_body,
        grid=(_GRID_A,),
        in_specs=[
            pl.BlockSpec((_EMB, _COLS_A), lambda i: (0, i)),
            pl.BlockSpec((_EMB, 1), lambda i: (0, 0)),
        ],
        out_specs=pl.BlockSpec((_COLS_A,), lambda i: (i,)),
        out_shape=jax.ShapeDtypeStruct((_VOCAB,), jnp.float32),
    )(table_t, w_col)


def _sc_pool_body(scores_hbm, ids_hbm, bvec_hbm, out_hbm,
                  idx0, idx1, vals0, vals1, out_v, b_v, semg, sema):
    wid = lax.axis_index("s") * _NC + lax.axis_index("c")
    base = wid * _COLS_W
    pltpu.sync_copy(bvec_hbm, b_v)
    bv = b_v[...]
    idx = [idx0, idx1]
    vals = [vals0, vals1]

    def assemble(ci, buf):
        col0 = base + ci * _CH

        def cp_issue(l, c2):
            pltpu.async_copy(ids_hbm.at[l, pl.ds(col0, _CH)],
                             buf.at[pl.ds(l * _CH, _CH)], sema)
            return c2

        def cp_drain(l, c2):
            pltpu.make_async_copy(ids_hbm.at[l, pl.ds(col0, _CH)],
                                  buf.at[pl.ds(l * _CH, _CH)], sema).wait()
            return c2

        lax.fori_loop(0, _L, cp_issue, 0)
        lax.fori_loop(0, _L, cp_drain, 0)

    assemble(0, idx[0])
    pltpu.async_copy(scores_hbm.at[idx[0]], vals[0], semg)
    assemble(1, idx[1])
    pltpu.async_copy(scores_hbm.at[idx[1]], vals[1], semg)
    for ci in range(_NCH):
        cur = ci % 2
        pltpu.make_async_copy(scores_hbm.at[idx[cur]], vals[cur], semg).wait()

        def red(l, accs):
            base_l = l * _CH
            return tuple(a + vals[cur][pl.ds(base_l + k * 16, 16)]
                         for k, a in enumerate(accs))

        zero = jnp.zeros((16,), jnp.float32)
        accs = lax.fori_loop(0, _L, red, (zero,) * (_CH // 16))
        for k, acc in enumerate(accs):
            z = acc + bv
            y = 1.0 / (1.0 + jnp.exp(-z))
            out_v[pl.ds(ci * _CH + k * 16, 16)] = y

        if ci + 2 < _NCH:
            assemble(ci + 2, idx[cur])
            pltpu.async_copy(scores_hbm.at[idx[cur]], vals[cur], semg)

    pltpu.sync_copy(out_v, out_hbm.at[pl.ds(base, _COLS_W)])


def _sc_pool(scores, ids_t, bvec):
    mesh = plsc.VectorSubcoreMesh(core_axis_name="c", subcore_axis_name="s")
    f = pl.kernel(
        _sc_pool_body,
        out_type=jax.ShapeDtypeStruct((_B,), jnp.float32),
        mesh=mesh,
        scratch_types=[
            pltpu.VMEM((_CHW,), jnp.int32),
            pltpu.VMEM((_CHW,), jnp.int32),
            pltpu.VMEM((_CHW,), jnp.float32),
            pltpu.VMEM((_CHW,), jnp.float32),
            pltpu.VMEM((_COLS_W,), jnp.float32),
            pltpu.VMEM((16,), jnp.float32),
            pltpu.SemaphoreType.DMA,
            pltpu.SemaphoreType.DMA,
        ],
    )
    return f(scores, ids_t, bvec)


def kernel(ids, table, W, b):
    table_t = table.astype(jnp.float32).T
    ids_t = ids.astype(jnp.int32).T
    w_col = W.astype(jnp.float32).reshape(_EMB, 1) * (1.0 / _L)
    scores = _scores(table_t, w_col)
    bvec = jnp.broadcast_to(b.astype(jnp.float32), (16,))
    out_flat = _sc_pool(scores, ids_t, bvec)
    return out_flat.reshape(_B, 1)


# speedup vs baseline: 1.0471x; 1.0091x over previous
import functools

import jax
import jax.numpy as jnp
from jax import lax
from jax.experimental import pallas as pl
from jax.experimental.pallas import tpu as pltpu
from jax.experimental.pallas import tpu_sc as plsc

_VOCAB = 1000000
_EMB = 32
_B = 16384
_L = 200
_PAD = 0

_COLS_A = 65536
_GRID_A = -(-_VOCAB // _COLS_A)

_NC, _NS = 2, 16
_NW = _NC * _NS
_COLS_W = _B // _NW
_CH = 128
_NCH = _COLS_W // _CH
_CHW = _L * _CH


def _score_body(x_ref, w_ref, out_ref):
    x = x_ref[...]
    w = w_ref[...]
    s = jnp.sum(x * w, axis=0, keepdims=True)
    i = pl.program_id(0)
    col = lax.broadcasted_iota(jnp.int32, (1, _COLS_A), 1)
    s = jnp.where((i == 0) & (col == _PAD), 0.0, s)
    out_ref[...] = s.reshape(_COLS_A)


def _scores(table_t, w_col):
    return pl.pallas_call(
        _score_body,
        grid=(_GRID_A,),
        in_specs=[
            pl.BlockSpec((_EMB, _COLS_A), lambda i: (0, i)),
            pl.BlockSpec((_EMB, 1), lambda i: (0, 0)),
        ],
        out_specs=pl.BlockSpec((_COLS_A,), lambda i: (i,)),
        out_shape=jax.ShapeDtypeStruct((_VOCAB,), jnp.float32),
    )(table_t, w_col)


def _sc_pool_body(scores_hbm, ids_hbm, bvec_hbm, out_hbm,
                  idx0, idx1, vals0, vals1, out_v, b_v, semg, sema):
    wid = lax.axis_index("s") * _NC + lax.axis_index("c")
    base = wid * _COLS_W
    pltpu.sync_copy(bvec_hbm, b_v)
    bv = b_v[...]
    idx = [idx0, idx1]
    vals = [vals0, vals1]

    def assemble(ci, buf):
        col0 = base + ci * _CH

        def cp_issue(l, c2):
            pltpu.async_copy(ids_hbm.at[l, pl.ds(col0, _CH)],
                             buf.at[pl.ds(l * _CH, _CH)], sema)
            return c2

        def cp_drain(l, c2):
            pltpu.make_async_copy(ids_hbm.at[l, pl.ds(col0, _CH)],
                                  buf.at[pl.ds(l * _CH, _CH)], sema).wait()
            return c2

        lax.fori_loop(0, _L, cp_issue, 0)
        lax.fori_loop(0, _L, cp_drain, 0)

    assemble(0, idx[0])
    pltpu.async_copy(scores_hbm.at[idx[0]], vals[0], semg)
    assemble(1, idx[1])
    for ci in range(_NCH):
        cur = ci % 2
        pltpu.make_async_copy(scores_hbm.at[idx[cur]], vals[cur], semg).wait()
        if ci + 1 < _NCH:
            pltpu.async_copy(scores_hbm.at[idx[1 - cur]], vals[1 - cur], semg)
        if ci + 2 < _NCH:
            assemble(ci + 2, idx[cur])

        def red(l, accs):
            base_l = l * _CH
            return tuple(a + vals[cur][pl.ds(base_l + k * 16, 16)]
                         for k, a in enumerate(accs))

        zero = jnp.zeros((16,), jnp.float32)
        accs = lax.fori_loop(0, _L, red, (zero,) * (_CH // 16))
        for k, acc in enumerate(accs):
            z = acc + bv
            y = 1.0 / (1.0 + jnp.exp(-z))
            out_v[pl.ds(ci * _CH + k * 16, 16)] = y

    pltpu.sync_copy(out_v, out_hbm.at[pl.ds(base, _COLS_W)])


def _sc_pool(scores, ids_t, bvec):
    mesh = plsc.VectorSubcoreMesh(core_axis_name="c", subcore_axis_name="s")
    f = pl.kernel(
        _sc_pool_body,
        out_type=jax.ShapeDtypeStruct((_B,), jnp.float32),
        mesh=mesh,
        scratch_types=[
            pltpu.VMEM((_CHW,), jnp.int32),
            pltpu.VMEM((_CHW,), jnp.int32),
            pltpu.VMEM((_CHW,), jnp.float32),
            pltpu.VMEM((_CHW,), jnp.float32),
            pltpu.VMEM((_COLS_W,), jnp.float32),
            pltpu.VMEM((16,), jnp.float32),
            pltpu.SemaphoreType.DMA,
            pltpu.SemaphoreType.DMA,
        ],
    )
    return f(scores, ids_t, bvec)


def kernel(ids, table, W, b):
    table_t = table.astype(jnp.float32).T
    ids_t = ids.astype(jnp.int32).T
    w_col = W.astype(jnp.float32).reshape(_EMB, 1) * (1.0 / _L)
    scores = _scores(table_t, w_col)
    bvec = jnp.broadcast_to(b.astype(jnp.float32), (16,))
    out_flat = _sc_pool(scores, ids_t, bvec)
    return out_flat.reshape(_B, 1)
